# baseline (speedup 1.0000x reference)
import functools

import jax
import jax.numpy as jnp
from jax import lax
from jax.experimental import pallas as pl
from jax.experimental.pallas import tpu as pltpu
from jax.experimental.pallas import tpu_sc as plsc

N = 10000
NP = 10240
E = 320000
D = 128
F = 16
NC = 2
NS = 16
NW = NC * NS
EPW = E // NW
CH = 80
NCHK = EPW // CH
NPT = NP // NS

_mesh = plsc.VectorSubcoreMesh(core_axis_name="c", subcore_axis_name="s")


def _zero_rows(rows_v):
  z = jnp.zeros((16,), jnp.float32)
  @pl.loop(0, CH)
  def _(i):
    for j in range(D // 16):
      rows_v[i, pl.ds(j * 16, 16)] = z


def _acc_zero_and_barrier(sid, zrow_v, acc, zsem):
  _zero_rows(zrow_v)
  base = sid * NPT
  for k in range(NPT // CH):
    pltpu.async_copy(zrow_v, acc.at[pl.ds(base + k * CH, CH)], zsem)
  for k in range(NPT // CH):
    pltpu.make_async_copy(zrow_v, acc.at[pl.ds(base + k * CH, CH)],
                          zsem).wait()
  plsc.subcore_barrier()


def _acc_writeback(cid, sid, rows2_v, acc, out_hbm, sem0, sem1):
  plsc.subcore_barrier()
  base = sid * NPT
  sems = (sem0, sem1)
  nkb = NPT // CH
  for k in range(nkb):
    b = k % 2
    if k >= 2:
      pltpu.make_async_copy(
          rows2_v.at[b], out_hbm.at[cid, pl.ds(base + (k - 2) * CH, CH)],
          sems[b]).wait()
    pltpu.sync_copy(acc.at[pl.ds(base + k * CH, CH)], rows2_v.at[b])
    pltpu.async_copy(
        rows2_v.at[b], out_hbm.at[cid, pl.ds(base + k * CH, CH)], sems[b])
  for k in (nkb - 2, nkb - 1):
    b = k % 2
    pltpu.make_async_copy(
        rows2_v.at[b], out_hbm.at[cid, pl.ds(base + k * CH, CH)],
        sems[b]).wait()


def _pipelined_accumulate(src_slice, didx_v, rows2_v, acc, sem0, sem1,
                          ssem0, ssem1):
  gsems = (sem0, sem1)
  ssems = (ssem0, ssem1)

  def fire_g(j, b):
    pltpu.async_copy(src_slice(j), rows2_v.at[b], gsems[b])

  def wait_g(j, b):
    pltpu.make_async_copy(src_slice(j), rows2_v.at[b], gsems[b]).wait()

  def fire_s(j, b):
    pltpu.async_copy(rows2_v.at[b], acc.at[didx_v.at[j]], ssems[b], add=True)

  def wait_s(j, b):
    pltpu.make_async_copy(rows2_v.at[b], acc.at[didx_v.at[j]],
                          ssems[b]).wait()

  @pl.loop(0, NCHK // 2)
  def _(g):
    c0 = 2 * g
    wait_g(c0, 0)
    fire_s(c0, 0)
    wait_g(c0 + 1, 1)
    fire_s(c0 + 1, 1)
    wait_s(c0, 0)
    fire_g(c0 + 2, 0)
    wait_s(c0 + 1, 1)

    @pl.when(c0 + 3 < NCHK)
    def _():
      fire_g(c0 + 3, 1)

  wait_g(NCHK - 1, 0)
  pltpu.sync_copy(rows2_v.at[0], acc.at[didx_v.at[NCHK - 1]], add=True)


@functools.partial(
    pl.kernel,
    out_type=jax.ShapeDtypeStruct((NC, NP, D), jnp.float32),
    mesh=_mesh,
    scratch_types=[
        pltpu.VMEM((NCHK, CH), jnp.int32),
        pltpu.VMEM((2, CH, D), jnp.float32),
        pltpu.VMEM_SHARED((NP, D), jnp.float32),
        pltpu.SemaphoreType.DMA,
        pltpu.SemaphoreType.DMA,
        pltpu.SemaphoreType.DMA,
        pltpu.SemaphoreType.DMA,
    ],
)
def _seg_sum_edges(rows_hbm, dst_hbm, out_hbm, didx_v, rows2_v, acc,
                   sem0, sem1, ssem0, ssem1):
  cid = lax.axis_index("c")
  sid = lax.axis_index("s")
  wid = cid * NS + sid
  ebase = wid * EPW
  src_slice = lambda j: rows_hbm.at[pl.ds(ebase + j * CH, CH)]
  pltpu.sync_copy(dst_hbm.at[wid], didx_v)
  pltpu.async_copy(src_slice(0), rows2_v.at[0], sem0)
  _acc_zero_and_barrier(sid, rows2_v.at[1], acc, ssem0)
  pltpu.async_copy(src_slice(1), rows2_v.at[1], sem1)
  _pipelined_accumulate(src_slice, didx_v, rows2_v, acc, sem0, sem1,
                        ssem0, ssem1)
  _acc_writeback(cid, sid, rows2_v, acc, out_hbm, sem0, sem1)


@functools.partial(
    pl.kernel,
    out_type=jax.ShapeDtypeStruct((NC, NP, D), jnp.float32),
    mesh=_mesh,
    scratch_types=[
        pltpu.VMEM((EPW,), jnp.int32),
        pltpu.VMEM((NCHK, CH), jnp.int32),
        pltpu.VMEM((2, CH, D), jnp.float32),
        pltpu.VMEM_SHARED((NP, D), jnp.float32),
        pltpu.SemaphoreType.DMA,
        pltpu.SemaphoreType.DMA,
        pltpu.SemaphoreType.DMA,
        pltpu.SemaphoreType.DMA,
    ],
)
def _gather_seg_sum(nodes_hbm, src_hbm, dst_hbm, out_hbm, sidx_v, didx_v,
                    rows2_v, acc, sem0, sem1, ssem0, ssem1):
  cid = lax.axis_index("c")
  sid = lax.axis_index("s")
  wid = cid * NS + sid
  src_slice = lambda j: nodes_hbm.at[sidx_v.at[pl.ds(j * CH, CH)]]
  pltpu.sync_copy(src_hbm.at[pl.ds(wid * EPW, EPW)], sidx_v)
  pltpu.sync_copy(dst_hbm.at[wid], didx_v)
  pltpu.async_copy(src_slice(0), rows2_v.at[0], sem0)
  _acc_zero_and_barrier(sid, rows2_v.at[1], acc, ssem0)
  pltpu.async_copy(src_slice(1), rows2_v.at[1], sem1)
  _pipelined_accumulate(src_slice, didx_v, rows2_v, acc, sem0, sem1,
                        ssem0, ssem1)
  _acc_writeback(cid, sid, rows2_v, acc, out_hbm, sem0, sem1)


BE = 2560


def _edge_linear_body(ef_ref, w_ref, out_ref):
  x = lax.dot_general(ef_ref[...], w_ref[...], (((1,), (0,)), ((), ())),
                      preferred_element_type=jnp.float32)
  out_ref[...] = jnp.maximum(x, 0.0)


def _static_body(p_ref, w_ref, static_ref, cur_ref):
  pool = p_ref[0] + p_ref[1]
  s = jnp.dot(pool, w_ref[...], preferred_element_type=jnp.float32)
  static_ref[...] = s
  cur_ref[...] = jnp.maximum(s, 0.0)


def _merge_body(static_ref, cur_ref, p_ref, w1_ref, w2_ref, out_ref):
  pool = p_ref[0] + p_ref[1]
  m = (static_ref[...]
       + jnp.dot(cur_ref[...], w1_ref[...], preferred_element_type=jnp.float32)
       + jnp.dot(pool, w2_ref[...], preferred_element_type=jnp.float32))
  out_ref[...] = jnp.maximum(m, 0.0)


BN = 2048


@jax.jit
def kernel(edge_feat, edge_index, W_e2l, W0, W1, W2):
  src = edge_index[0]
  dst = edge_index[1].reshape(NW, NCHK, CH)
  w_e = W_e2l.T
  w0 = W0.T
  w1 = W1.T
  w2 = W2.T

  edge_msg = pl.pallas_call(
      _edge_linear_body,
      grid=(E // BE,),
      in_specs=[
          pl.BlockSpec((BE, F), lambda i: (i, 0)),
          pl.BlockSpec((F, D), lambda i: (0, 0)),
      ],
      out_specs=pl.BlockSpec((BE, D), lambda i: (i, 0)),
      out_shape=jax.ShapeDtypeStruct((E, D), jnp.float32),
  )(edge_feat, w_e)

  part = _seg_sum_edges(edge_msg, dst)

  static, cur = pl.pallas_call(
      _static_body,
      grid=(NP // BN,),
      in_specs=[
          pl.BlockSpec((NC, BN, D), lambda i: (0, i, 0)),
          pl.BlockSpec((D, D), lambda i: (0, 0)),
      ],
      out_specs=[
          pl.BlockSpec((BN, D), lambda i: (i, 0)),
          pl.BlockSpec((BN, D), lambda i: (i, 0)),
      ],
      out_shape=[
          jax.ShapeDtypeStruct((NP, D), jnp.float32),
          jax.ShapeDtypeStruct((NP, D), jnp.float32),
      ],
  )(part, w0)

  merge = pl.pallas_call(
      _merge_body,
      grid=(NP // BN,),
      in_specs=[
          pl.BlockSpec((BN, D), lambda i: (i, 0)),
          pl.BlockSpec((BN, D), lambda i: (i, 0)),
          pl.BlockSpec((NC, BN, D), lambda i: (0, i, 0)),
          pl.BlockSpec((D, D), lambda i: (0, 0)),
          pl.BlockSpec((D, D), lambda i: (0, 0)),
      ],
      out_specs=pl.BlockSpec((BN, D), lambda i: (i, 0)),
      out_shape=jax.ShapeDtypeStruct((NP, D), jnp.float32),
  )

  for _ in range(2):
    pool = _gather_seg_sum(cur, src, dst)
    cur = merge(static, cur, pool, w1, w2)
  return cur[:N]


# speedup vs baseline: 1.1708x; 1.1708x over previous
import functools

import jax
import jax.numpy as jnp
from jax import lax
from jax.experimental import pallas as pl
from jax.experimental.pallas import tpu as pltpu
from jax.experimental.pallas import tpu_sc as plsc

N = 10000
NP = 10240
E = 320000
D = 128
F = 16
NC = 2
NS = 16
NW = NC * NS
EPW = E // NW
CH = 80
NCHK = EPW // CH
NPT = NP // NS

_mesh = plsc.VectorSubcoreMesh(core_axis_name="c", subcore_axis_name="s")


def _zero_rows(rows_v):
  z = jnp.zeros((16,), jnp.float32)
  @pl.loop(0, CH)
  def _(i):
    for j in range(D // 16):
      rows_v[i, pl.ds(j * 16, 16)] = z


def _acc_zero_and_barrier(sid, zrow_v, acc, zsem):
  _zero_rows(zrow_v)
  base = sid * NPT
  for k in range(NPT // CH):
    pltpu.async_copy(zrow_v, acc.at[pl.ds(base + k * CH, CH)], zsem)
  for k in range(NPT // CH):
    pltpu.make_async_copy(zrow_v, acc.at[pl.ds(base + k * CH, CH)],
                          zsem).wait()
  plsc.subcore_barrier()


def _acc_writeback(cid, sid, rows2_v, acc, out_hbm, sem0, sem1):
  plsc.subcore_barrier()
  base = sid * NPT
  sems = (sem0, sem1)
  nkb = NPT // CH
  for k in range(nkb):
    b = k % 2
    if k >= 2:
      pltpu.make_async_copy(
          rows2_v.at[b], out_hbm.at[cid, pl.ds(base + (k - 2) * CH, CH)],
          sems[b]).wait()
    pltpu.sync_copy(acc.at[pl.ds(base + k * CH, CH)], rows2_v.at[b])
    pltpu.async_copy(
        rows2_v.at[b], out_hbm.at[cid, pl.ds(base + k * CH, CH)], sems[b])
  for k in (nkb - 2, nkb - 1):
    b = k % 2
    pltpu.make_async_copy(
        rows2_v.at[b], out_hbm.at[cid, pl.ds(base + k * CH, CH)],
        sems[b]).wait()


def _pipelined_accumulate(src_slice, didx_v, rows2_v, acc, sem0, sem1,
                          ssem0, ssem1):
  gsems = (sem0, sem1)
  ssems = (ssem0, ssem1)

  def fire_g(j, b):
    pltpu.async_copy(src_slice(j), rows2_v.at[b], gsems[b])

  def wait_g(j, b):
    pltpu.make_async_copy(src_slice(j), rows2_v.at[b], gsems[b]).wait()

  def fire_s(j, b):
    pltpu.async_copy(rows2_v.at[b], acc.at[didx_v.at[j]], ssems[b], add=True)

  def wait_s(j, b):
    pltpu.make_async_copy(rows2_v.at[b], acc.at[didx_v.at[j]],
                          ssems[b]).wait()

  @pl.loop(0, NCHK // 2)
  def _(g):
    c0 = 2 * g
    wait_g(c0, 0)
    fire_s(c0, 0)
    wait_s(c0, 0)
    fire_g(c0 + 2, 0)
    wait_g(c0 + 1, 1)
    fire_s(c0 + 1, 1)
    wait_s(c0 + 1, 1)

    @pl.when(c0 + 3 < NCHK)
    def _():
      fire_g(c0 + 3, 1)

  wait_g(NCHK - 1, 0)
  pltpu.sync_copy(rows2_v.at[0], acc.at[didx_v.at[NCHK - 1]], add=True)


@functools.partial(
    pl.kernel,
    out_type=jax.ShapeDtypeStruct((NC, NP, D), jnp.float32),
    mesh=_mesh,
    scratch_types=[
        pltpu.VMEM((NCHK, CH), jnp.int32),
        pltpu.VMEM((2, CH, D), jnp.float32),
        pltpu.VMEM_SHARED((NP, D), jnp.float32),
        pltpu.SemaphoreType.DMA,
        pltpu.SemaphoreType.DMA,
        pltpu.SemaphoreType.DMA,
        pltpu.SemaphoreType.DMA,
    ],
)
def _seg_sum_edges(rows_hbm, dst_hbm, out_hbm, didx_v, rows2_v, acc,
                   sem0, sem1, ssem0, ssem1):
  cid = lax.axis_index("c")
  sid = lax.axis_index("s")
  wid = cid * NS + sid
  ebase = wid * EPW
  src_slice = lambda j: rows_hbm.at[pl.ds(ebase + j * CH, CH)]
  pltpu.sync_copy(dst_hbm.at[wid], didx_v)
  pltpu.async_copy(src_slice(0), rows2_v.at[0], sem0)
  _acc_zero_and_barrier(sid, rows2_v.at[1], acc, ssem0)
  pltpu.async_copy(src_slice(1), rows2_v.at[1], sem1)
  _pipelined_accumulate(src_slice, didx_v, rows2_v, acc, sem0, sem1,
                        ssem0, ssem1)
  _acc_writeback(cid, sid, rows2_v, acc, out_hbm, sem0, sem1)


@functools.partial(
    pl.kernel,
    out_type=jax.ShapeDtypeStruct((NC, NP, D), jnp.float32),
    mesh=_mesh,
    scratch_types=[
        pltpu.VMEM((EPW,), jnp.int32),
        pltpu.VMEM((NCHK, CH), jnp.int32),
        pltpu.VMEM((2, CH, D), jnp.float32),
        pltpu.VMEM_SHARED((NP, D), jnp.float32),
        pltpu.SemaphoreType.DMA,
        pltpu.SemaphoreType.DMA,
        pltpu.SemaphoreType.DMA,
        pltpu.SemaphoreType.DMA,
    ],
)
def _gather_seg_sum(nodes_hbm, src_hbm, dst_hbm, out_hbm, sidx_v, didx_v,
                    rows2_v, acc, sem0, sem1, ssem0, ssem1):
  cid = lax.axis_index("c")
  sid = lax.axis_index("s")
  wid = cid * NS + sid
  src_slice = lambda j: nodes_hbm.at[sidx_v.at[pl.ds(j * CH, CH)]]
  pltpu.sync_copy(src_hbm.at[pl.ds(wid * EPW, EPW)], sidx_v)
  pltpu.sync_copy(dst_hbm.at[wid], didx_v)
  pltpu.async_copy(src_slice(0), rows2_v.at[0], sem0)
  _acc_zero_and_barrier(sid, rows2_v.at[1], acc, ssem0)
  pltpu.async_copy(src_slice(1), rows2_v.at[1], sem1)
  _pipelined_accumulate(src_slice, didx_v, rows2_v, acc, sem0, sem1,
                        ssem0, ssem1)
  _acc_writeback(cid, sid, rows2_v, acc, out_hbm, sem0, sem1)


BE = 2560


def _edge_linear_body(ef_ref, w_ref, out_ref):
  x = lax.dot_general(ef_ref[...], w_ref[...], (((1,), (0,)), ((), ())),
                      preferred_element_type=jnp.float32)
  out_ref[...] = jnp.maximum(x, 0.0)


def _static_body(p_ref, w_ref, static_ref, cur_ref):
  pool = p_ref[0] + p_ref[1]
  s = jnp.dot(pool, w_ref[...], preferred_element_type=jnp.float32)
  static_ref[...] = s
  cur_ref[...] = jnp.maximum(s, 0.0)


def _merge_body(static_ref, cur_ref, p_ref, w1_ref, w2_ref, out_ref):
  pool = p_ref[0] + p_ref[1]
  m = (static_ref[...]
       + jnp.dot(cur_ref[...], w1_ref[...], preferred_element_type=jnp.float32)
       + jnp.dot(pool, w2_ref[...], preferred_element_type=jnp.float32))
  out_ref[...] = jnp.maximum(m, 0.0)


BN = 2048


@jax.jit
def kernel(edge_feat, edge_index, W_e2l, W0, W1, W2):
  src = edge_index[0]
  dst = edge_index[1].reshape(NW, NCHK, CH)
  w_e = W_e2l.T
  w0 = W0.T
  w1 = W1.T
  w2 = W2.T

  edge_msg = pl.pallas_call(
      _edge_linear_body,
      grid=(E // BE,),
      in_specs=[
          pl.BlockSpec((BE, F), lambda i: (i, 0)),
          pl.BlockSpec((F, D), lambda i: (0, 0)),
      ],
      out_specs=pl.BlockSpec((BE, D), lambda i: (i, 0)),
      out_shape=jax.ShapeDtypeStruct((E, D), jnp.float32),
  )(edge_feat, w_e)

  part = _seg_sum_edges(edge_msg, dst)

  static, cur = pl.pallas_call(
      _static_body,
      grid=(NP // BN,),
      in_specs=[
          pl.BlockSpec((NC, BN, D), lambda i: (0, i, 0)),
          pl.BlockSpec((D, D), lambda i: (0, 0)),
      ],
      out_specs=[
          pl.BlockSpec((BN, D), lambda i: (i, 0)),
          pl.BlockSpec((BN, D), lambda i: (i, 0)),
      ],
      out_shape=[
          jax.ShapeDtypeStruct((NP, D), jnp.float32),
          jax.ShapeDtypeStruct((NP, D), jnp.float32),
      ],
  )(part, w0)

  merge = pl.pallas_call(
      _merge_body,
      grid=(NP // BN,),
      in_specs=[
          pl.BlockSpec((BN, D), lambda i: (i, 0)),
          pl.BlockSpec((BN, D), lambda i: (i, 0)),
          pl.BlockSpec((NC, BN, D), lambda i: (0, i, 0)),
          pl.BlockSpec((D, D), lambda i: (0, 0)),
          pl.BlockSpec((D, D), lambda i: (0, 0)),
      ],
      out_specs=pl.BlockSpec((BN, D), lambda i: (i, 0)),
      out_shape=jax.ShapeDtypeStruct((NP, D), jnp.float32),
  )

  for _ in range(2):
    pool = _gather_seg_sum(cur, src, dst)
    cur = merge(static, cur, pool, w1, w2)
  return cur[:N]
